# Initial kernel scaffold; baseline (speedup 1.0000x reference)
import jax
import jax.numpy as jnp
from jax.experimental import pallas as pl


def kernel(mood, raga, taal, tempo, duration, mood_table, raga_table, taal_table, W_tempo, b_tempo, W_dur, b_dur, W_proj, b_proj, ln_gamma, ln_beta):
    raise NotImplementedError("write your pallas kernel here")


# speedup vs baseline: 4.5564x; 4.5564x over previous
import functools

import jax
import jax.numpy as jnp
from jax import lax
from jax.experimental import pallas as pl
from jax.experimental.pallas import tpu as pltpu
from jax.experimental.pallas import tpu_sc as plsc

B = 16384
D_MODEL = 1024

_NC = 2
_NS = 16
_NW = _NC * _NS
_CH = 128
_BUF = 512


def _make_sc_gather(batch):
    rows_per_w = batch // _NW
    n_ch = rows_per_w // _CH
    n_outer = max(1, rows_per_w // _BUF)
    ch_per_outer = n_ch // n_outer
    mesh = plsc.VectorSubcoreMesh(core_axis_name="c", subcore_axis_name="s")

    @functools.partial(
        pl.kernel,
        mesh=mesh,
        out_type=jax.ShapeDtypeStruct((batch, 128), jnp.float32),
        scratch_types=[
            pltpu.VMEM((n_ch, _CH), jnp.int32),
            pltpu.VMEM((min(rows_per_w, _BUF), 128), jnp.float32),
            pltpu.SemaphoreType.DMA,
        ],
    )
    def gather_kernel(table_hbm, idx_hbm, out_hbm, idx_v, rows_v, sem):
        wid = lax.axis_index("s") * _NC + lax.axis_index("c")
        pltpu.sync_copy(idx_hbm.at[pl.ds(wid * n_ch, n_ch)], idx_v)
        for o in range(n_outer):
            copies = []
            for j in range(ch_per_outer):
                copies.append(
                    pltpu.async_copy(
                        table_hbm.at[idx_v.at[o * ch_per_outer + j]],
                        rows_v.at[pl.ds(j * _CH, _CH)],
                        sem,
                    )
                )
            for c in copies:
                c.wait()
            pltpu.sync_copy(
                rows_v.at[pl.ds(0, ch_per_outer * _CH)],
                out_hbm.at[pl.ds(wid * rows_per_w + o * ch_per_outer * _CH,
                                 ch_per_outer * _CH)],
            )

    return gather_kernel


def _tc_body(mt_ref, r_ref, tp_ref, du_ref,
             Wp_ref, bp_ref, Wt_ref, bt_ref, Wd_ref, bd_ref,
             g_ref, be_ref, o_ref):
    tp = tp_ref[...] * Wt_ref[...] + bt_ref[...]
    du = du_ref[...] * Wd_ref[...] + bd_ref[...]
    mt = mt_ref[...]
    cond = jnp.concatenate(
        [mt[:_BLK, :64], r_ref[...], mt[_BLK:, :64], tp, du], axis=1)
    h = jnp.dot(cond, Wp_ref[...], preferred_element_type=jnp.float32) + bp_ref[...]
    h = h * 0.5 * (1.0 + lax.erf(h * (2.0 ** -0.5)))
    mu = jnp.mean(h, axis=1, keepdims=True)
    var = jnp.mean((h - mu) ** 2, axis=1, keepdims=True)
    o_ref[...] = (h - mu) / jnp.sqrt(var + 1e-5) * g_ref[...] + be_ref[...]


_BLK = 512


def _tc_forward(mt, r, tempo, duration, W_proj, b_proj,
                W_tempo, b_tempo, W_dur, b_dur, ln_gamma, ln_beta):
    grid = (B // _BLK,)
    nblk = B // _BLK
    const = lambda s: pl.BlockSpec(s, lambda i: (0, 0))
    return pl.pallas_call(
        _tc_body,
        grid=grid,
        in_specs=[
            pl.BlockSpec((2 * _BLK, 128), lambda i: (i, 0)),
            pl.BlockSpec((_BLK, 128), lambda i: (i, 0)),
            pl.BlockSpec((_BLK, 1), lambda i: (i, 0)),
            pl.BlockSpec((_BLK, 1), lambda i: (i, 0)),
            const((320, D_MODEL)), const((1, D_MODEL)),
            const((1, 32)), const((1, 32)), const((1, 32)), const((1, 32)),
            const((1, D_MODEL)), const((1, D_MODEL)),
        ],
        out_specs=pl.BlockSpec((_BLK, D_MODEL), lambda i: (i, 0)),
        out_shape=jax.ShapeDtypeStruct((B, D_MODEL), jnp.float32),
        compiler_params=pltpu.CompilerParams(
            dimension_semantics=("arbitrary",),
        ),
    )(mt, r, tempo, duration, W_proj, b_proj,
      W_tempo, b_tempo, W_dur, b_dur, ln_gamma, ln_beta)


def kernel(mood, raga, taal, tempo, duration,
           mood_table, raga_table, taal_table,
           W_tempo, b_tempo, W_dur, b_dur,
           W_proj, b_proj, ln_gamma, ln_beta):
    nm = mood_table.shape[0]
    small_table = jnp.concatenate([
        jnp.pad(mood_table, ((0, 0), (0, 64))),
        jnp.pad(taal_table, ((0, 0), (0, 64))),
    ], axis=0)
    mood_i = mood.astype(jnp.int32).reshape(B // _BLK, _BLK)
    taal_i = taal.astype(jnp.int32).reshape(B // _BLK, _BLK) + nm
    small_idx = jnp.stack([mood_i, taal_i], axis=1).reshape(2 * B // _CH, _CH)
    raga_i = raga.astype(jnp.int32).reshape(B // _CH, _CH)

    mt = _make_sc_gather(2 * B)(small_table, small_idx)
    r = _make_sc_gather(B)(raga_table, raga_i)

    return _tc_forward(
        mt, r,
        tempo.reshape(B, 1), duration.reshape(B, 1),
        W_proj, b_proj.reshape(1, D_MODEL),
        W_tempo, b_tempo.reshape(1, 32), W_dur, b_dur.reshape(1, 32),
        ln_gamma.reshape(1, D_MODEL), ln_beta.reshape(1, D_MODEL),
    )
